# baseline (speedup 1.0000x reference)
import functools

import jax
import jax.numpy as jnp
from jax import lax
from jax.experimental import pallas as pl
from jax.experimental.pallas import tpu as pltpu
from jax.experimental.pallas import tpu_sc as plsc

VOCAB = 1000000
B = 4096
L = 200
DIM = 64
LANES = 16

_info = plsc.get_sparse_core_info()
NC = _info.num_cores
NS = _info.num_subcores
NW = NC * NS
EPW = B // NW
TOK = EPW * L
C1 = 104
C2 = 96
LPAD = 208
NBUF = 8
RB = 8000

_mesh = plsc.VectorSubcoreMesh(core_axis_name="c", subcore_axis_name="s")

_DNUMS = lax.GatherDimensionNumbers(
    offset_dims=(), collapsed_slice_dims=(0,), start_index_map=(0,))


def _permute(x, idx):
    return lax.gather(x, idx[:, None], _DNUMS, (1,),
                      mode=lax.GatherScatterMode.PROMISE_IN_BOUNDS)


def _tv_body(table_ref, w_ref, out_ref):
    out_ref[...] = jnp.dot(table_ref[...], w_ref[...],
                           preferred_element_type=jnp.float32)


_tv_call = pl.pallas_call(
    _tv_body,
    grid=(VOCAB // RB,),
    in_specs=[
        pl.BlockSpec((RB, DIM), lambda i: (i, 0)),
        pl.BlockSpec((DIM, 1), lambda i: (0, 0)),
    ],
    out_specs=pl.BlockSpec((RB, 1), lambda i: (i, 0)),
    out_shape=jax.ShapeDtypeStruct((VOCAB, 1), jnp.float32),
)


@functools.partial(
    pl.kernel,
    mesh=_mesh,
    compiler_params=pltpu.CompilerParams(use_tc_tiling_on_sc=False),
    out_type=jax.ShapeDtypeStruct((B,), jnp.float32),
    scratch_types=[
        pltpu.VMEM((TOK,), jnp.int32),
        pltpu.VMEM((EPW,), jnp.int32),
        pltpu.VMEM((LANES,), jnp.float32),
        pltpu.VMEM((NBUF, LPAD), jnp.float32),
        pltpu.VMEM((EPW,), jnp.float32),
        pltpu.SemaphoreType.DMA,
        pltpu.SemaphoreType.DMA,
        pltpu.SemaphoreType.DMA,
        pltpu.SemaphoreType.DMA,
        pltpu.SemaphoreType.DMA,
        pltpu.SemaphoreType.DMA,
        pltpu.SemaphoreType.DMA,
        pltpu.SemaphoreType.DMA,
    ],
)
def _gather_pool(tflat_hbm, lens_hbm, tv_hbm, b_hbm, out_hbm,
                 tidx, len_v, b_v, ring, out_v,
                 sem0, sem1, sem2, sem3, sem4, sem5, sem6, sem7):
    wid = lax.axis_index("s") * NC + lax.axis_index("c")
    base = pl.multiple_of(wid * EPW, 8)
    sems = (sem0, sem1, sem2, sem3, sem4, sem5, sem6, sem7)

    pltpu.sync_copy(tflat_hbm.at[pl.ds(base * L, TOK)], tidx)
    pltpu.sync_copy(lens_hbm.at[pl.ds(base, EPW)], len_v)
    pltpu.sync_copy(b_hbm, b_v)

    bv = b_v[...]
    lane = lax.broadcasted_iota(jnp.int32, (LANES,), 0)
    zerov = jnp.zeros((LANES,), jnp.float32)

    def issue(e, u, sem):
        off = pl.multiple_of(e * L, 8)
        pltpu.async_copy(
            tv_hbm.at[tidx.at[pl.ds(off, C1)]],
            ring.at[u, pl.ds(0, C1)], sem)
        pltpu.async_copy(
            tv_hbm.at[tidx.at[pl.ds(off + C1, C2)]],
            ring.at[u, pl.ds(C1, C2)], sem)

    def drain(u, sem):
        pltpu.make_async_copy(
            tv_hbm.at[pl.ds(0, L)], ring.at[u, pl.ds(0, L)], sem).wait()

    def process(e, u, cur):
        e16 = e % LANES
        g0 = pl.multiple_of(e - e16, LANES)
        lv = len_v[pl.ds(g0, LANES)]
        lsplat = _permute(lv, jnp.full((LANES,), e16, jnp.int32))
        s = zerov
        for k in range(LPAD // LANES):
            v = ring[u, pl.ds(k * LANES, LANES)]
            pos = lane + (k * LANES)
            s = s + jnp.where(pos < lsplat, v, zerov)
        for sh in (8, 4, 2, 1):
            s = s + _permute(s, lane ^ sh)
        cur = jnp.where(lane == e16, s, cur)

        @pl.when(e16 == LANES - 1)
        def _():
            lg = len_v[pl.ds(g0, LANES)].astype(jnp.float32)
            out_v[pl.ds(g0, LANES)] = cur / lg + bv

        return cur

    for u in range(NBUF):
        issue(u, u, sems[u])

    def body(i, cur):
        for u in range(NBUF):
            e = i * NBUF + u
            drain(u, sems[u])
            cur = process(e, u, cur)

            @pl.when(e + NBUF < EPW)
            def _():
                issue(e + NBUF, u, sems[u])
        return cur

    lax.fori_loop(0, EPW // NBUF, body, zerov)
    pltpu.sync_copy(out_v, out_hbm.at[pl.ds(base, EPW)])


def kernel(text, text_lengths, table, W, b):
    tv = _tv_call(table, W.astype(jnp.float32)).reshape(VOCAB)
    tflat = text.astype(jnp.int32).reshape(-1)
    lens = text_lengths.astype(jnp.int32)
    b16 = jnp.broadcast_to(b.astype(jnp.float32), (LANES,))
    out = _gather_pool(tflat, lens, tv, b16)
    return out.reshape(B, 1)


# speedup vs baseline: 1.4053x; 1.4053x over previous
import functools

import jax
import jax.numpy as jnp
from jax import lax
from jax.experimental import pallas as pl
from jax.experimental.pallas import tpu as pltpu
from jax.experimental.pallas import tpu_sc as plsc

VOCAB = 1000000
B = 4096
L = 200
DIM = 64
LANES = 16

_info = plsc.get_sparse_core_info()
NC = _info.num_cores
NS = _info.num_subcores
NW = NC * NS
EPW = B // NW
TOK = EPW * L
C1 = 104
C2 = 96
LPAD = 208
NBUF = 8
RB = 8192

_mesh = plsc.VectorSubcoreMesh(core_axis_name="c", subcore_axis_name="s")

_DNUMS = lax.GatherDimensionNumbers(
    offset_dims=(), collapsed_slice_dims=(0,), start_index_map=(0,))


def _permute(x, idx):
    return lax.gather(x, idx[:, None], _DNUMS, (1,),
                      mode=lax.GatherScatterMode.PROMISE_IN_BOUNDS)


def _tv_body(w_ref, table_ref, out_ref):
    out_ref[...] = lax.dot_general(
        w_ref[...], table_ref[...],
        dimension_numbers=(((1,), (1,)), ((), ())),
        preferred_element_type=jnp.float32)


_tv_call = pl.pallas_call(
    _tv_body,
    grid=((VOCAB + RB - 1) // RB,),
    in_specs=[
        pl.BlockSpec((1, DIM), lambda i: (0, 0)),
        pl.BlockSpec((RB, DIM), lambda i: (i, 0)),
    ],
    out_specs=pl.BlockSpec((1, RB), lambda i: (0, i)),
    out_shape=jax.ShapeDtypeStruct((1, VOCAB), jnp.float32),
)


@functools.partial(
    pl.kernel,
    mesh=_mesh,
    compiler_params=pltpu.CompilerParams(use_tc_tiling_on_sc=False),
    out_type=jax.ShapeDtypeStruct((B,), jnp.float32),
    scratch_types=[
        pltpu.VMEM((TOK,), jnp.int32),
        pltpu.VMEM((EPW,), jnp.int32),
        pltpu.VMEM((LANES,), jnp.float32),
        pltpu.VMEM((NBUF, LPAD), jnp.float32),
        pltpu.VMEM((EPW,), jnp.float32),
        pltpu.SemaphoreType.DMA,
        pltpu.SemaphoreType.DMA,
        pltpu.SemaphoreType.DMA,
        pltpu.SemaphoreType.DMA,
        pltpu.SemaphoreType.DMA,
        pltpu.SemaphoreType.DMA,
        pltpu.SemaphoreType.DMA,
        pltpu.SemaphoreType.DMA,
    ],
)
def _gather_pool(tflat_hbm, lens_hbm, tv_hbm, b_hbm, out_hbm,
                 tidx, len_v, b_v, ring, out_v,
                 sem0, sem1, sem2, sem3, sem4, sem5, sem6, sem7):
    wid = lax.axis_index("s") * NC + lax.axis_index("c")
    base = pl.multiple_of(wid * EPW, 8)
    sems = (sem0, sem1, sem2, sem3, sem4, sem5, sem6, sem7)

    pltpu.sync_copy(tflat_hbm.at[pl.ds(base * L, TOK)], tidx)
    pltpu.sync_copy(lens_hbm.at[pl.ds(base, EPW)], len_v)
    pltpu.sync_copy(b_hbm, b_v)

    bv = b_v[...]
    lane = lax.broadcasted_iota(jnp.int32, (LANES,), 0)
    zerov = jnp.zeros((LANES,), jnp.float32)

    def issue(e, u, sem):
        off = pl.multiple_of(e * L, 8)
        pltpu.async_copy(
            tv_hbm.at[tidx.at[pl.ds(off, C1)]],
            ring.at[u, pl.ds(0, C1)], sem)
        pltpu.async_copy(
            tv_hbm.at[tidx.at[pl.ds(off + C1, C2)]],
            ring.at[u, pl.ds(C1, C2)], sem)

    def drain(u, sem):
        pltpu.make_async_copy(
            tv_hbm.at[pl.ds(0, L)], ring.at[u, pl.ds(0, L)], sem).wait()

    def process(e, u, cur):
        e16 = e % LANES
        g0 = pl.multiple_of(e - e16, LANES)
        lv = len_v[pl.ds(g0, LANES)]
        lsplat = _permute(lv, jnp.full((LANES,), e16, jnp.int32))
        s = zerov
        for k in range(LPAD // LANES):
            v = ring[u, pl.ds(k * LANES, LANES)]
            pos = lane + (k * LANES)
            s = s + jnp.where(pos < lsplat, v, zerov)
        for sh in (8, 4, 2, 1):
            s = s + _permute(s, lane ^ sh)
        cur = jnp.where(lane == e16, s, cur)

        @pl.when(e16 == LANES - 1)
        def _():
            lg = len_v[pl.ds(g0, LANES)].astype(jnp.float32)
            out_v[pl.ds(g0, LANES)] = cur / lg + bv

        return cur

    for u in range(NBUF):
        issue(u, u, sems[u])

    def body(i, cur):
        for u in range(NBUF):
            e = i * NBUF + u
            drain(u, sems[u])
            cur = process(e, u, cur)

            @pl.when(e + NBUF < EPW)
            def _():
                issue(e + NBUF, u, sems[u])
        return cur

    lax.fori_loop(0, EPW // NBUF, body, zerov)
    pltpu.sync_copy(out_v, out_hbm.at[pl.ds(base, EPW)])


def kernel(text, text_lengths, table, W, b):
    tv = _tv_call(W.astype(jnp.float32).reshape(1, DIM), table).reshape(VOCAB)
    tflat = text.astype(jnp.int32).reshape(-1)
    lens = text_lengths.astype(jnp.int32)
    b16 = jnp.broadcast_to(b.astype(jnp.float32), (LANES,))
    out = _gather_pool(tflat, lens, tv, b16)
    return out.reshape(B, 1)
